# baseline (speedup 1.0000x reference)
import functools

import jax
import jax.numpy as jnp
from jax import lax
from jax.experimental import pallas as pl
from jax.experimental.pallas import tpu as pltpu
from jax.experimental.pallas import tpu_sc as plsc

B = 1024
D = 64
DP = 2 * D
H = 200
NC = 2
NS = 16
NW = NC * NS
BPW = B // NW
H0 = 104
H1 = H - H0
NLANE = 16
DV = D // NLANE


def _sc_gather_pool(users_r, items_r, ur_idx_r, ir_idx_r,
                    u2_r, i2_r, big2_r,
                    ue_out, ie_out, ure_out, ire_out,
                    idx_v, kidx_v, rows_v, hidx_v, hrows0_v, hrows1_v,
                    pooled_v, sem):
    wid = lax.axis_index("s") * NC + lax.axis_index("c")
    base = wid * BPW

    def gather_simple(src_idx_hbm, tbl2_r, out_hbm):
        pltpu.sync_copy(src_idx_hbm.at[pl.ds(base, BPW)], idx_v)
        for j in range(BPW // NLANE):
            v = idx_v[pl.ds(j * NLANE, NLANE)]
            kidx_v[pl.ds(j * NLANE, NLANE)] = jax.lax.shift_right_logical(
                v, jnp.int32(1))
        pltpu.async_copy(tbl2_r.at[kidx_v], rows_v, sem).wait()
        for b in range(BPW):
            vj = idx_v[pl.ds((b // NLANE) * NLANE, NLANE)]
            off = (vj[b % NLANE] & jnp.int32(1)) * jnp.int32(D)
            for k in range(DV):
                pooled_v[b, pl.ds(k * NLANE, NLANE)] = (
                    rows_v[b, pl.ds(off + k * NLANE, NLANE)])
        pltpu.sync_copy(pooled_v, out_hbm.at[pl.ds(base, BPW)])

    gather_simple(users_r, u2_r, ue_out)
    gather_simple(items_r, i2_r, ie_out)

    def pool_table(hist_idx_hbm, out_hbm, lane_off):
        pltpu.sync_copy(hist_idx_hbm.at[pl.ds(base * H, BPW * H)], hidx_v)
        bufs = (hrows0_v, hrows1_v)

        def fire(b):
            buf = bufs[b % 2]
            c0 = pltpu.async_copy(
                big2_r.at[hidx_v.at[pl.ds(b * H, H0)]],
                buf.at[pl.ds(0, H0)], sem)
            c1 = pltpu.async_copy(
                big2_r.at[hidx_v.at[pl.ds(b * H + H0, H1)]],
                buf.at[pl.ds(H0, H1)], sem)
            return (c0, c1)

        inflight = fire(0)
        for b in range(BPW):
            for c in inflight:
                c.wait()
            buf = bufs[b % 2]
            if b + 1 < BPW:
                inflight = fire(b + 1)

            def body(j, acc):
                ea, ob = acc
                ea = tuple(ea[k] + buf[j, pl.ds(lane_off + k * NLANE, NLANE)]
                           for k in range(DV))
                ob = tuple(ob[k] + buf[j + 1, pl.ds(lane_off + k * NLANE, NLANE)]
                           for k in range(DV))
                return (ea, ob)

            zero = jnp.zeros((NLANE,), jnp.float32)
            init = (tuple(zero for _ in range(DV)),
                    tuple(zero for _ in range(DV)))
            ea, ob = plsc.parallel_loop(0, H, step=2, unroll=4,
                                        carry=init)(body)
            scale = jnp.float32(1.0 / H)
            for k in range(DV):
                pooled_v[b, pl.ds(lane_off + k * NLANE, NLANE)] = (
                    (ea[k] + ob[k]) * scale)
        pltpu.sync_copy(pooled_v, out_hbm.at[pl.ds(base, BPW)])

    pool_table(ur_idx_r, ure_out, 0)
    pool_table(ir_idx_r, ire_out, D)


@functools.partial(
    pl.kernel,
    out_type=tuple(jax.ShapeDtypeStruct((B, DP), jnp.float32)
                   for _ in range(4)),
    mesh=plsc.VectorSubcoreMesh(core_axis_name="c", subcore_axis_name="s"),
    scratch_types=[
        pltpu.VMEM((BPW,), jnp.int32),
        pltpu.VMEM((BPW,), jnp.int32),
        pltpu.VMEM((BPW, DP), jnp.float32),
        pltpu.VMEM((BPW * H,), jnp.int32),
        pltpu.VMEM((H, DP), jnp.float32),
        pltpu.VMEM((H, DP), jnp.float32),
        pltpu.VMEM((BPW, DP), jnp.float32),
        pltpu.SemaphoreType.DMA,
    ],
    compiler_params=pltpu.CompilerParams(use_tc_tiling_on_sc=True),
)
def _sc_kernel(*refs):
    _sc_gather_pool(*refs)


def _tc_combine(uei_ref, iei_ref, urp_ref, irp_ref,
                wui_ref, wur_ref, wri_ref, wrr_ref,
                total_ref, regs_ref):
    ue = uei_ref[:, 0:D]
    ie = iei_ref[:, 0:D]
    ure = urp_ref[:, 0:D]
    ire = irp_ref[:, D:DP]
    inf_ui = jnp.dot(ue * ie, wui_ref[...], preferred_element_type=jnp.float32)
    inf_ur = jnp.dot(ue + ire, wur_ref[...], preferred_element_type=jnp.float32)
    inf_ri = jnp.dot(jnp.maximum(ure, ie), wri_ref[...],
                     preferred_element_type=jnp.float32)
    inf_rr = (jnp.dot(ure, wrr_ref[0:D, :], preferred_element_type=jnp.float32)
              + jnp.dot(ire, wrr_ref[D:2 * D, :],
                        preferred_element_type=jnp.float32))
    total_ref[...] = 0.25 * (inf_ui + inf_ur + inf_ri + inf_rr)
    regs = 0.001 * (jnp.sqrt(jnp.sum(ue * ue)) + jnp.sqrt(jnp.sum(ie * ie)))
    regs_ref[...] = regs.reshape(1, 1)


_tc_combine_call = pl.pallas_call(
    _tc_combine,
    out_shape=(jax.ShapeDtypeStruct((B, 1), jnp.float32),
               jax.ShapeDtypeStruct((1, 1), jnp.float32)),
)


def kernel(users, items, users_ratings, items_ratings,
           users_sparse_ratings, items_sparse_ratings,
           UsersEmb, ItemsEmb, UsersRatingsEmb, ItemsRatingsEmb,
           W_ui, W_ur, W_ri, W_rr):
    del users_sparse_ratings, items_sparse_ratings
    users = users.astype(jnp.int32)
    items = items.astype(jnp.int32)
    users_ratings = users_ratings.astype(jnp.int32).reshape(B * H)
    items_ratings = items_ratings.astype(jnp.int32).reshape(B * H)
    u2 = UsersEmb.reshape(UsersEmb.shape[0] // 2, DP)
    i2 = ItemsEmb.reshape(ItemsEmb.shape[0] // 2, DP)
    big2 = jnp.concatenate([UsersRatingsEmb, ItemsRatingsEmb], axis=1)
    uei, iei, urp, irp = _sc_kernel(
        users, items, users_ratings, items_ratings, u2, i2, big2)
    total, regs = _tc_combine_call(uei, iei, urp, irp, W_ui, W_ur, W_ri, W_rr)
    return total, regs[0, 0]
